# baseline (speedup 1.0000x reference)
import functools

import jax
import jax.numpy as jnp
from jax.experimental import pallas as pl
from jax.experimental.pallas import tpu as pltpu

_IN_SIZES = (120, 120, 120, 120, 120, 96, 56, 32)
_N_IN = len(_IN_SIZES)
_IN_OFFS = tuple(sum(_IN_SIZES[:k]) for k in range(_N_IN))
_N_OUT = 28
_OUT_P = 28


def _se_kernel(x_ref, w1_ref, w2_ref, o_ref, slab_ref, acc_ref, gate_ref,
               w1v_ref, w2v_ref, in_sems, w_sems, *, inv_hw):
    i = pl.program_id(0)

    @pl.when(i == 0)
    def _issue():
        for k in range(_N_IN):
            pltpu.make_async_copy(
                x_ref.at[pl.ds(_IN_OFFS[k], _IN_SIZES[k])],
                slab_ref.at[pl.ds(_IN_OFFS[k], _IN_SIZES[k])],
                in_sems.at[k]).start()
        pltpu.make_async_copy(w1_ref, w1v_ref, w_sems.at[0]).start()
        pltpu.make_async_copy(w2_ref, w2v_ref, w_sems.at[1]).start()

    for k in range(_N_IN):
        @pl.when(i == k)
        def _pool(k=k):
            pltpu.make_async_copy(
                slab_ref.at[pl.ds(_IN_OFFS[k], _IN_SIZES[k])],
                slab_ref.at[pl.ds(_IN_OFFS[k], _IN_SIZES[k])],
                in_sems.at[k]).wait()
            s = jnp.sum(slab_ref[pl.ds(_IN_OFFS[k], _IN_SIZES[k])],
                        axis=0)
            if k == 0:
                acc_ref[...] = s
            else:
                acc_ref[...] += s

    @pl.when(i == _N_IN)
    def _gates():
        pltpu.make_async_copy(w1v_ref, w1v_ref, w_sems.at[0]).wait()
        pltpu.make_async_copy(w2v_ref, w2v_ref, w_sems.at[1]).wait()
        y1 = jax.lax.dot_general(
            acc_ref[...] * inv_hw, w1v_ref[...], (((1,), (1,)), ((), ())),
            preferred_element_type=jnp.float32)
        y1 = jnp.maximum(y1, 0.0)
        y2 = jax.lax.dot_general(
            y1, w2v_ref[...], (((1,), (1,)), ((), ())),
            preferred_element_type=jnp.float32)
        gate_ref[...] = 1.0 / (1.0 + jnp.exp(-y2))

    @pl.when(i >= _N_IN)
    def _scale():
        j = i - _N_IN
        o_ref[...] = slab_ref[pl.ds(j * _OUT_P, _OUT_P)] * gate_ref[...]


def kernel(x, w1, w2):
    B, C, H, W = x.shape
    Cr = w1.shape[0]
    HW = H * W
    xv = x.transpose(2, 3, 0, 1).reshape(HW, B, C)

    out = pl.pallas_call(
        functools.partial(_se_kernel, inv_hw=1.0 / float(HW)),
        out_shape=jax.ShapeDtypeStruct((HW, B, C), x.dtype),
        grid=(_N_IN + _N_OUT,),
        in_specs=[
            pl.BlockSpec(memory_space=pl.ANY),
            pl.BlockSpec(memory_space=pl.ANY),
            pl.BlockSpec(memory_space=pl.ANY),
        ],
        out_specs=pl.BlockSpec(
            (_OUT_P, B, C), lambda i: (jnp.maximum(i - _N_IN, 0), 0, 0)),
        scratch_shapes=[
            pltpu.VMEM((HW, B, C), jnp.float32),
            pltpu.VMEM((B, C), jnp.float32),
            pltpu.VMEM((B, C), jnp.float32),
            pltpu.VMEM((Cr, C), jnp.float32),
            pltpu.VMEM((C, Cr), jnp.float32),
            pltpu.SemaphoreType.DMA((_N_IN,)),
            pltpu.SemaphoreType.DMA((2,)),
        ],
        compiler_params=pltpu.CompilerParams(
            dimension_semantics=("arbitrary",),
            vmem_limit_bytes=63 << 20),
    )(xv, w1, w2)
    return out.reshape(H, W, B, C).transpose(2, 3, 0, 1)


# speedup vs baseline: 1.0522x; 1.0522x over previous
import functools

import jax
import jax.numpy as jnp
from jax.experimental import pallas as pl
from jax.experimental.pallas import tpu as pltpu

_IN_SIZES = (160, 160, 160, 128, 96, 48, 32)
_N_IN = len(_IN_SIZES)
_IN_OFFS = tuple(sum(_IN_SIZES[:k]) for k in range(_N_IN))
_N_OUT = 16
_OUT_P = 49


def _se_kernel(x_ref, w1_ref, w2_ref, o_ref, slab_ref, acc_ref, gate_ref,
               w1v_ref, w2v_ref, in_sems, w_sems, *, inv_hw):
    i = pl.program_id(0)

    @pl.when(i == 0)
    def _issue():
        for k in range(_N_IN):
            pltpu.make_async_copy(
                x_ref.at[pl.ds(_IN_OFFS[k], _IN_SIZES[k])],
                slab_ref.at[pl.ds(_IN_OFFS[k], _IN_SIZES[k])],
                in_sems.at[k]).start()
        pltpu.make_async_copy(w1_ref, w1v_ref, w_sems.at[0]).start()
        pltpu.make_async_copy(w2_ref, w2v_ref, w_sems.at[1]).start()

    for k in range(_N_IN):
        @pl.when(i == k)
        def _pool(k=k):
            pltpu.make_async_copy(
                slab_ref.at[pl.ds(_IN_OFFS[k], _IN_SIZES[k])],
                slab_ref.at[pl.ds(_IN_OFFS[k], _IN_SIZES[k])],
                in_sems.at[k]).wait()
            s = jnp.sum(slab_ref[pl.ds(_IN_OFFS[k], _IN_SIZES[k])],
                        axis=0)
            if k == 0:
                acc_ref[...] = s
            else:
                acc_ref[...] += s

    @pl.when(i == _N_IN)
    def _gates():
        pltpu.make_async_copy(w1v_ref, w1v_ref, w_sems.at[0]).wait()
        pltpu.make_async_copy(w2v_ref, w2v_ref, w_sems.at[1]).wait()
        y1 = jax.lax.dot_general(
            acc_ref[...] * inv_hw, w1v_ref[...], (((1,), (1,)), ((), ())),
            preferred_element_type=jnp.float32)
        y1 = jnp.maximum(y1, 0.0)
        y2 = jax.lax.dot_general(
            y1, w2v_ref[...], (((1,), (1,)), ((), ())),
            preferred_element_type=jnp.float32)
        gate_ref[...] = 1.0 / (1.0 + jnp.exp(-y2))

    @pl.when(i >= _N_IN)
    def _scale():
        j = i - _N_IN
        o_ref[...] = slab_ref[pl.ds(j * _OUT_P, _OUT_P)] * gate_ref[...]


def kernel(x, w1, w2):
    B, C, H, W = x.shape
    Cr = w1.shape[0]
    HW = H * W
    xv = x.transpose(2, 3, 0, 1).reshape(HW, B, C)

    out = pl.pallas_call(
        functools.partial(_se_kernel, inv_hw=1.0 / float(HW)),
        out_shape=jax.ShapeDtypeStruct((HW, B, C), x.dtype),
        grid=(_N_IN + _N_OUT,),
        in_specs=[
            pl.BlockSpec(memory_space=pl.ANY),
            pl.BlockSpec(memory_space=pl.ANY),
            pl.BlockSpec(memory_space=pl.ANY),
        ],
        out_specs=pl.BlockSpec(
            (_OUT_P, B, C), lambda i: (jnp.maximum(i - _N_IN, 0), 0, 0)),
        scratch_shapes=[
            pltpu.VMEM((HW, B, C), jnp.float32),
            pltpu.VMEM((B, C), jnp.float32),
            pltpu.VMEM((B, C), jnp.float32),
            pltpu.VMEM((Cr, C), jnp.float32),
            pltpu.VMEM((C, Cr), jnp.float32),
            pltpu.SemaphoreType.DMA((_N_IN,)),
            pltpu.SemaphoreType.DMA((2,)),
        ],
        compiler_params=pltpu.CompilerParams(
            dimension_semantics=("arbitrary",),
            vmem_limit_bytes=63 << 20),
    )(xv, w1, w2)
    return out.reshape(H, W, B, C).transpose(2, 3, 0, 1)


# speedup vs baseline: 1.0988x; 1.0444x over previous
import functools

import jax
import jax.numpy as jnp
from jax.experimental import pallas as pl
from jax.experimental.pallas import tpu as pltpu

_IN_SIZES = (120, 120, 120, 120, 120, 96, 56, 32)
_N_IN = len(_IN_SIZES)
_IN_OFFS = tuple(sum(_IN_SIZES[:k]) for k in range(_N_IN))
_N_OUT = 16
_OUT_P = 49


def _se_kernel(x_ref, w1_ref, w2_ref, o_ref, slab_ref, acc_ref, gate_ref,
               w1v_ref, w2v_ref, in_sems, w_sems, *, inv_hw):
    i = pl.program_id(0)

    @pl.when(i == 0)
    def _issue():
        for k in range(_N_IN):
            pltpu.make_async_copy(
                x_ref.at[pl.ds(_IN_OFFS[k], _IN_SIZES[k])],
                slab_ref.at[pl.ds(_IN_OFFS[k], _IN_SIZES[k])],
                in_sems.at[k]).start()
        pltpu.make_async_copy(w1_ref, w1v_ref, w_sems.at[0]).start()
        pltpu.make_async_copy(w2_ref, w2v_ref, w_sems.at[1]).start()

    for k in range(_N_IN):
        @pl.when(i == k)
        def _pool(k=k):
            pltpu.make_async_copy(
                slab_ref.at[pl.ds(_IN_OFFS[k], _IN_SIZES[k])],
                slab_ref.at[pl.ds(_IN_OFFS[k], _IN_SIZES[k])],
                in_sems.at[k]).wait()
            s = jnp.sum(slab_ref[pl.ds(_IN_OFFS[k], _IN_SIZES[k])],
                        axis=0)
            if k == 0:
                acc_ref[...] = s
            else:
                acc_ref[...] += s

    @pl.when(i == _N_IN)
    def _gates():
        pltpu.make_async_copy(w1v_ref, w1v_ref, w_sems.at[0]).wait()
        pltpu.make_async_copy(w2v_ref, w2v_ref, w_sems.at[1]).wait()
        y1 = jax.lax.dot_general(
            acc_ref[...] * inv_hw, w1v_ref[...], (((1,), (1,)), ((), ())),
            preferred_element_type=jnp.float32)
        y1 = jnp.maximum(y1, 0.0)
        y2 = jax.lax.dot_general(
            y1, w2v_ref[...], (((1,), (1,)), ((), ())),
            preferred_element_type=jnp.float32)
        gate_ref[...] = 1.0 / (1.0 + jnp.exp(-y2))

    @pl.when(i >= _N_IN)
    def _scale():
        j = i - _N_IN
        o_ref[...] = slab_ref[pl.ds(j * _OUT_P, _OUT_P)] * gate_ref[...]


def kernel(x, w1, w2):
    B, C, H, W = x.shape
    Cr = w1.shape[0]
    HW = H * W
    xv = x.transpose(2, 3, 0, 1).reshape(HW, B, C)

    out = pl.pallas_call(
        functools.partial(_se_kernel, inv_hw=1.0 / float(HW)),
        out_shape=jax.ShapeDtypeStruct((HW, B, C), x.dtype),
        grid=(_N_IN + _N_OUT,),
        in_specs=[
            pl.BlockSpec(memory_space=pl.ANY),
            pl.BlockSpec(memory_space=pl.ANY),
            pl.BlockSpec(memory_space=pl.ANY),
        ],
        out_specs=pl.BlockSpec(
            (_OUT_P, B, C), lambda i: (jnp.maximum(i - _N_IN, 0), 0, 0)),
        scratch_shapes=[
            pltpu.VMEM((HW, B, C), jnp.float32),
            pltpu.VMEM((B, C), jnp.float32),
            pltpu.VMEM((B, C), jnp.float32),
            pltpu.VMEM((Cr, C), jnp.float32),
            pltpu.VMEM((C, Cr), jnp.float32),
            pltpu.SemaphoreType.DMA((_N_IN,)),
            pltpu.SemaphoreType.DMA((2,)),
        ],
        compiler_params=pltpu.CompilerParams(
            dimension_semantics=("arbitrary",),
            vmem_limit_bytes=63 << 20),
    )(xv, w1, w2)
    return out.reshape(H, W, B, C).transpose(2, 3, 0, 1)
